# baseline (speedup 1.0000x reference)
import functools

import jax
import jax.numpy as jnp
from jax import lax
from jax.experimental import pallas as pl
from jax.experimental.pallas import tpu as pltpu
from jax.experimental.pallas import tpu_sc as plsc

EMBED_DIM = 64
ROWPAD = 128
NC = 2
NS = 16
NW = NC * NS
CHUNK = 128
LANES = 16
DSUB = 8
TBLK = 2048


def _table_widen(table_t):
    d, vocab = table_t.shape
    grid = (vocab + TBLK - 1) // TBLK

    def body(in_ref, o_ref):
        o_ref[:, pl.ds(0, d)] = in_ref[...].T

    return pl.pallas_call(
        body,
        grid=(grid,),
        in_specs=[pl.BlockSpec((d, TBLK), lambda i: (0, i))],
        out_specs=pl.BlockSpec((TBLK, ROWPAD), lambda i: (i, 0)),
        out_shape=jax.ShapeDtypeStruct((vocab, ROWPAD), jnp.float32),
    )(table_t)


def _gather_call(batch: int, fields: int):
    b_per_w = batch // NW
    tpw = b_per_w // CHUNK
    nbt = batch // CHUNK
    ndt = EMBED_DIM // DSUB
    mesh = plsc.VectorSubcoreMesh(core_axis_name="c", subcore_axis_name="s")

    @functools.partial(
        pl.kernel,
        mesh=mesh,
        out_type=jax.ShapeDtypeStruct(
            (fields, ndt, nbt, DSUB, CHUNK), jnp.float32),
        scratch_types=[
            pltpu.VMEM((fields, b_per_w), jnp.int32),
            pltpu.VMEM((CHUNK // LANES, LANES), jnp.int32),
            pltpu.VMEM((EMBED_DIM, LANES), jnp.int32),
            [pltpu.VMEM((CHUNK, ROWPAD), jnp.float32) for _ in range(tpw)],
            [pltpu.VMEM((ndt, DSUB, CHUNK), jnp.float32) for _ in range(tpw)],
            [pltpu.SemaphoreType.DMA for _ in range(tpw)],
            [pltpu.SemaphoreType.DMA for _ in range(tpw)],
        ],
        compiler_params=pltpu.CompilerParams(
            use_tc_tiling_on_sc=False, needs_layout_passes=False),
    )
    def k(idx_hbm, rvec_hbm, cvec_hbm, table_hbm, out_hbm,
          slab, rvs, cvs, rows, tbuf, gsem, wsem):
        wid = lax.axis_index("s") * NC + lax.axis_index("c")
        pltpu.sync_copy(idx_hbm.at[:, pl.ds(wid * b_per_w, b_per_w)], slab)
        pltpu.sync_copy(rvec_hbm, rvs)
        pltpu.sync_copy(cvec_hbm, cvs)

        def gather(t, f):
            return pltpu.make_async_copy(
                table_hbm.at[slab.at[f, pl.ds(t * CHUNK, CHUNK)]],
                rows[t], gsem[t])

        def writeback(t, f, dt):
            return pltpu.make_async_copy(
                tbuf[t].at[dt],
                out_hbm.at[f, dt, wid * tpw + t], wsem[t])

        def transpose(t):
            def dtbody(dt, _):
                for ds in range(DSUB):
                    cvec = cvs[dt * DSUB + ds, :]
                    for g in range(CHUNK // LANES):
                        vals = plsc.load_gather(rows[t], [rvs[g, :], cvec])
                        tbuf[t][dt, ds, pl.ds(g * LANES, LANES)] = vals
                return 0

            lax.fori_loop(0, ndt, dtbody, 0)

        def step(t, f, first, last):
            gather(t, f).wait()
            if not first:
                for dt in range(ndt):
                    writeback(t, f - 1, dt).wait()
            transpose(t)
            if not last:
                gather(t, f + 1).start()
            for dt in range(ndt):
                writeback(t, f, dt).start()

        for t in range(tpw):
            gather(t, 0).start()
        for t in range(tpw):
            step(t, 0, True, False)

        def body(f, _):
            for t in range(tpw):
                step(t, f, False, False)
            return 0

        lax.fori_loop(1, fields - 1, body, 0)

        for t in range(tpw):
            step(t, fields - 1, False, True)
        for t in range(tpw):
            for dt in range(ndt):
                writeback(t, fields - 1, dt).wait()

    return k


def kernel(nodes, table):
    batch, fields = nodes.shape
    nodes_t = nodes.T
    table_w = _table_widen(table.T)
    rvec = jnp.arange(CHUNK, dtype=jnp.int32).reshape(CHUNK // LANES, LANES)
    cvec = jnp.broadcast_to(
        jnp.arange(EMBED_DIM, dtype=jnp.int32)[:, None], (EMBED_DIM, LANES))
    out5 = _gather_call(batch, fields)(nodes_t, rvec, cvec, table_w)
    return out5.transpose(2, 4, 0, 1, 3).reshape(batch, fields, EMBED_DIM)


# speedup vs baseline: 1.7191x; 1.7191x over previous
import functools

import jax
import jax.numpy as jnp
from jax import lax
from jax.experimental import pallas as pl
from jax.experimental.pallas import tpu as pltpu
from jax.experimental.pallas import tpu_sc as plsc

EMBED_DIM = 64
ROWPAD = 128
NC = 2
NS = 16
NW = NC * NS
CHUNK = 128
LANES = 16
DSUB = 8
TBLK = 2048


def _table_widen(table_t):
    d, vocab = table_t.shape
    grid = (vocab + TBLK - 1) // TBLK

    def body(in_ref, o_ref):
        o_ref[:, pl.ds(0, d)] = in_ref[...].T

    return pl.pallas_call(
        body,
        grid=(grid,),
        in_specs=[pl.BlockSpec((d, TBLK), lambda i: (0, i))],
        out_specs=pl.BlockSpec((TBLK, ROWPAD), lambda i: (i, 0)),
        out_shape=jax.ShapeDtypeStruct((vocab, ROWPAD), jnp.float32),
    )(table_t)


def _gather_call(batch: int, fields: int):
    b_per_w = batch // NW
    tpw = b_per_w // CHUNK
    nbt = batch // CHUNK
    ndt = EMBED_DIM // DSUB
    mesh = plsc.VectorSubcoreMesh(core_axis_name="c", subcore_axis_name="s")

    @functools.partial(
        pl.kernel,
        mesh=mesh,
        out_type=jax.ShapeDtypeStruct(
            (fields, ndt, nbt, DSUB, CHUNK), jnp.float32),
        scratch_types=[
            pltpu.VMEM((fields, b_per_w), jnp.int32),
            pltpu.VMEM((CHUNK // LANES, LANES), jnp.int32),
            pltpu.VMEM((EMBED_DIM, LANES), jnp.int32),
            [pltpu.VMEM((CHUNK, ROWPAD), jnp.float32) for _ in range(tpw)],
            [pltpu.VMEM((ndt, DSUB, CHUNK), jnp.float32) for _ in range(tpw)],
            [pltpu.SemaphoreType.DMA for _ in range(tpw)],
            [pltpu.SemaphoreType.DMA for _ in range(tpw)],
        ],
        compiler_params=pltpu.CompilerParams(
            use_tc_tiling_on_sc=False, needs_layout_passes=False),
    )
    def k(idx_hbm, rvec_hbm, cvec_hbm, table_hbm, out_hbm,
          slab, rvs, cvs, rows, tbuf, gsem, wsem):
        wid = lax.axis_index("s") * NC + lax.axis_index("c")
        pltpu.sync_copy(idx_hbm.at[:, pl.ds(wid * b_per_w, b_per_w)], slab)
        pltpu.sync_copy(rvec_hbm, rvs)
        pltpu.sync_copy(cvec_hbm, cvs)

        def gather(t, f):
            return pltpu.make_async_copy(
                table_hbm.at[slab.at[f, pl.ds(t * CHUNK, CHUNK)]],
                rows[t], gsem[t])

        def writeback(t, f, dt):
            return pltpu.make_async_copy(
                tbuf[t].at[dt],
                out_hbm.at[f, dt, wid * tpw + t], wsem[t])

        def transpose(t):
            @plsc.parallel_loop(0, EMBED_DIM, unroll=4)
            def cbody(c):
                dt = c // DSUB
                ds = lax.rem(c, DSUB)
                cvec = cvs[c, :]
                for g in range(CHUNK // LANES):
                    vals = plsc.load_gather(rows[t], [rvs[g, :], cvec])
                    tbuf[t][dt, ds, pl.ds(g * LANES, LANES)] = vals

        def step(t, f):
            gather(t, f).wait()

            @pl.when(f > 0)
            def _():
                for dt in range(ndt):
                    writeback(t, f - 1, dt).wait()

            transpose(t)

            @pl.when(f < fields - 1)
            def _():
                gather(t, f + 1).start()

            for dt in range(ndt):
                writeback(t, f, dt).start()

        for t in range(tpw):
            gather(t, 0).start()

        def body(f, _):
            for t in range(tpw):
                step(t, f)
            return 0

        lax.fori_loop(0, fields, body, 0)

        for t in range(tpw):
            for dt in range(ndt):
                writeback(t, fields - 1, dt).wait()

    return k


def kernel(nodes, table):
    batch, fields = nodes.shape
    nodes_t = nodes.T
    table_w = _table_widen(table.T)
    rvec = jnp.arange(CHUNK, dtype=jnp.int32).reshape(CHUNK // LANES, LANES)
    cvec = jnp.broadcast_to(
        jnp.arange(EMBED_DIM, dtype=jnp.int32)[:, None], (EMBED_DIM, LANES))
    out5 = _gather_call(batch, fields)(nodes_t, rvec, cvec, table_w)
    return out5.transpose(2, 4, 0, 1, 3).reshape(batch, fields, EMBED_DIM)
